# baseline (speedup 1.0000x reference)
import functools

import jax
import jax.numpy as jnp
from jax import lax
from jax.experimental import pallas as pl
from jax.experimental.pallas import tpu as pltpu
from jax.experimental.pallas import tpu_sc as plsc

N = 10000
E = 320000
IN = 128
HID = 128
CLS = 40

NC = 2
NS = 16
NW = NC * NS
EW = E // NW
CHUNK = 80
NCHUNK = EW // CHUNK
RPT = N // NS
ZROWS = 25


def _fill(ref, rows, width, value):
    vec = jnp.full((16,), value, dtype=jnp.float32)
    per_row = width // 16

    def body(t, _):
        r = t // per_row
        col = (t % per_row) * 16
        ref[r, pl.ds(col, 16)] = vec
        return _

    lax.fori_loop(0, rows * per_row, body, None)


def _sc_mesh():
    return plsc.VectorSubcoreMesh(core_axis_name="c", subcore_axis_name="s",
                                  num_cores=NC, num_subcores=NS)


def _sc_params():
    return pltpu.CompilerParams(use_tc_tiling_on_sc=False)


def _make_sc_agg(feat_width, out_width=None, interpret=False):
    scratch = [
        pltpu.VMEM((NCHUNK, CHUNK), jnp.int32),
        pltpu.VMEM((NCHUNK, CHUNK), jnp.int32),
        pltpu.VMEM((CHUNK, feat_width), jnp.float32),
        pltpu.VMEM((CHUNK, feat_width), jnp.float32),
        pltpu.VMEM((CHUNK, feat_width), jnp.float32),
        pltpu.VMEM_SHARED((N, feat_width), jnp.float32),
    ] + [pltpu.SemaphoreType.DMA] * 6
    stacked = out_width is not None
    oshape = (N, out_width) if stacked else (NC * N, feat_width)

    @functools.partial(
        pl.kernel,
        out_type=jax.ShapeDtypeStruct(oshape, jnp.float32),
        mesh=_sc_mesh(),
        scratch_types=scratch,
        compiler_params=_sc_params(),
        interpret=interpret,
    )
    def sc_agg(p_hbm, e_hbm, out_hbm, src_v, dst_v, rows0, rows1, rows2,
               acc, gsem0, gsem1, gsem2, ssem0, ssem1, ssem2):
        c = lax.axis_index("c")
        s = lax.axis_index("s")
        w = c * NS + s
        rbase = s * RPT

        pltpu.sync_copy(e_hbm.at[0, pl.ds(w * NCHUNK, NCHUNK)], src_v)
        pltpu.sync_copy(e_hbm.at[1, pl.ds(w * NCHUNK, NCHUNK)], dst_v)

        _fill(rows0, CHUNK, feat_width, 0.0)
        for i in range(RPT // CHUNK):
            pltpu.sync_copy(rows0, acc.at[pl.ds(rbase + i * CHUNK, CHUNK)])
        rem = RPT % CHUNK
        pltpu.sync_copy(rows0.at[pl.ds(0, rem)],
                        acc.at[pl.ds(rbase + RPT - rem, rem)])
        plsc.subcore_barrier()

        bufs = (rows0, rows1, rows2)
        gsems = (gsem0, gsem1, gsem2)
        ssems = (ssem0, ssem1, ssem2)

        def gather(j, b):
            pltpu.async_copy(p_hbm.at[src_v.at[j]], bufs[b], gsems[b])

        def gwait(b):
            pltpu.make_async_copy(p_hbm.at[src_v.at[0]], bufs[b],
                                  gsems[b]).wait()

        def scat(j, b):
            pltpu.async_copy(bufs[b], acc.at[dst_v.at[j]], ssems[b], add=True)

        def swait(b):
            pltpu.make_async_copy(bufs[b], acc.at[dst_v.at[0]],
                                  ssems[b]).wait()

        def slot(j, b, wait_prev, prefetch):
            nxt = (b + 1) % 3
            if wait_prev:
                swait(nxt)
            if prefetch:
                gather(j + 1, nxt)
            gwait(b)
            scat(j, b)

        gather(0, 0)
        slot(0, 0, False, True)
        slot(1, 1, False, True)

        def body(i, _):
            slot(3 * i + 2, 2, True, True)
            slot(3 * i + 3, 0, True, True)
            slot(3 * i + 4, 1, True, True)
            return _

        lax.fori_loop(0, (NCHUNK - 5) // 3, body, None)
        slot(NCHUNK - 3, (NCHUNK - 3) % 3, True, True)
        slot(NCHUNK - 2, (NCHUNK - 2) % 3, True, True)
        slot(NCHUNK - 1, (NCHUNK - 1) % 3, True, False)
        swait((NCHUNK - 2) % 3)
        swait((NCHUNK - 1) % 3)
        plsc.subcore_barrier()

        if stacked:
            pltpu.sync_copy(
                acc.at[pl.ds(rbase, RPT)],
                out_hbm.at[pl.ds(rbase, RPT), pl.ds(c * feat_width, feat_width)])
        else:
            pltpu.sync_copy(acc.at[pl.ds(rbase, RPT)],
                            out_hbm.at[pl.ds(c * N + rbase, RPT)])

    return sc_agg


def _make_sc_deg(interpret=False):
    scratch = [
        pltpu.VMEM((NCHUNK, CHUNK), jnp.int32),
        pltpu.VMEM((CHUNK, 16), jnp.float32),
        pltpu.VMEM((ZROWS, 16), jnp.float32),
        pltpu.VMEM_SHARED((N, 16), jnp.float32),
        pltpu.SemaphoreType.DMA,
    ]

    @functools.partial(
        pl.kernel,
        out_type=jax.ShapeDtypeStruct((N, 128), jnp.float32),
        mesh=_sc_mesh(),
        scratch_types=scratch,
        compiler_params=_sc_params(),
        interpret=interpret,
    )
    def sc_deg(e_hbm, deg_hbm, dst_v, ones_v, zbuf16, dacc, sem):
        c = lax.axis_index("c")
        s = lax.axis_index("s")
        w = c * NS + s
        rbase = s * RPT

        pltpu.sync_copy(e_hbm.at[1, pl.ds(w * NCHUNK, NCHUNK)], dst_v)
        _fill(ones_v, CHUNK, 16, 1.0)
        _fill(zbuf16, ZROWS, 16, 0.0)

        def zero_body(i, _):
            pltpu.sync_copy(zbuf16, dacc.at[pl.ds(rbase + i * ZROWS, ZROWS)])
            return _

        lax.fori_loop(0, RPT // ZROWS, zero_body, None)
        plsc.subcore_barrier()

        GRP = 5

        def body(i, _):
            def fire(g, _):
                pltpu.async_copy(ones_v, dacc.at[dst_v.at[i * GRP + g]], sem,
                                 add=True)
                return _

            lax.fori_loop(0, GRP, fire, None)

            def drain(g, _):
                pltpu.make_async_copy(ones_v, dacc.at[dst_v.at[0]], sem).wait()
                return _

            lax.fori_loop(0, GRP, drain, None)
            return _

        lax.fori_loop(0, NCHUNK // GRP, body, None)
        plsc.subcore_barrier()

        pltpu.sync_copy(dacc.at[pl.ds(rbase, RPT)],
                        deg_hbm.at[pl.ds(rbase, RPT), pl.ds(c * 16, 16)])

    return sc_deg


_make_sc_agg = functools.lru_cache(maxsize=None)(_make_sc_agg)
_make_sc_deg = functools.lru_cache(maxsize=None)(_make_sc_deg)

BN = 2000


def _row_spec(width):
    return pl.BlockSpec((BN, width), lambda i: (i, 0))


def _half_spec(width, half):
    return pl.BlockSpec((BN, width), lambda i: (half * (N // BN) + i, 0))


def _full_spec(shape):
    return pl.BlockSpec(shape, lambda i: tuple(0 for _ in shape))


def _mm_p_body(x_ref, wn_ref, p_ref):
    p_ref[...] = jnp.dot(x_ref[...], wn_ref[...],
                         preferred_element_type=jnp.float32)


def _mm_p(x, wn, interpret=False):
    return pl.pallas_call(
        _mm_p_body,
        grid=(N // BN,),
        in_specs=[_row_spec(HID), _full_spec((HID, HID))],
        out_specs=_row_spec(HID),
        out_shape=jax.ShapeDtypeStruct((N, HID), jnp.float32),
        interpret=interpret,
    )(x, wn)


def _mm_s_body(x_ref, ws_ref, b_ref, s_ref):
    s_ref[...] = (jnp.dot(x_ref[...], ws_ref[...],
                          preferred_element_type=jnp.float32) + b_ref[...])


def _mm_s(x, ws, b, width, interpret=False):
    return pl.pallas_call(
        _mm_s_body,
        grid=(N // BN,),
        in_specs=[_row_spec(HID), _full_spec((HID, width)),
                  _full_spec((1, width))],
        out_specs=_row_spec(width),
        out_shape=jax.ShapeDtypeStruct((N, width), jnp.float32),
        interpret=interpret,
    )(x, ws, b)


def _h_of(sp_ref, a0_ref, a1_ref, inv):
    return jnp.maximum(sp_ref[...] + (a0_ref[...] + a1_ref[...]) * inv, 0.0)


def _inv_of(d_ref):
    deg = d_ref[...][:, :1] + d_ref[...][:, 16:17]
    return 1.0 / jnp.maximum(deg, 1.0)


def _mm_mid_body(sp_ref, a0_ref, a1_ref, d_ref, wn_ref, p_ref, inv_ref):
    inv = _inv_of(d_ref)
    h = _h_of(sp_ref, a0_ref, a1_ref, inv)
    p_ref[...] = jnp.dot(h, wn_ref[...], preferred_element_type=jnp.float32)
    inv_ref[...] = jnp.broadcast_to(inv, (BN, 16))


def _mm_mid(s_prev, agg, deg, wn, interpret=False):
    return pl.pallas_call(
        _mm_mid_body,
        grid=(N // BN,),
        in_specs=[_row_spec(HID), _half_spec(HID, 0), _half_spec(HID, 1),
                  _row_spec(128), _full_spec((HID, HID))],
        out_specs=[_row_spec(HID), _row_spec(16)],
        out_shape=[jax.ShapeDtypeStruct((N, HID), jnp.float32),
                   jax.ShapeDtypeStruct((N, 16), jnp.float32)],
        interpret=interpret,
    )(s_prev, agg, agg, deg, wn)


def _mm_mid_s_body(sp_ref, a0_ref, a1_ref, d_ref, ws_ref, b_ref, s_ref):
    h = _h_of(sp_ref, a0_ref, a1_ref, _inv_of(d_ref))
    s_ref[...] = (jnp.dot(h, ws_ref[...], preferred_element_type=jnp.float32)
                  + b_ref[...])


def _mm_mid_s(s_prev, agg, deg, ws, b, interpret=False):
    return pl.pallas_call(
        _mm_mid_s_body,
        grid=(N // BN,),
        in_specs=[_row_spec(HID), _half_spec(HID, 0), _half_spec(HID, 1),
                  _row_spec(128), _full_spec((HID, HID)),
                  _full_spec((1, HID))],
        out_specs=_row_spec(HID),
        out_shape=jax.ShapeDtypeStruct((N, HID), jnp.float32),
        interpret=interpret,
    )(s_prev, agg, agg, deg, ws, b)


def _mm_last_body(sp_ref, a0_ref, a1_ref, inv_ref, wn_ref, p_ref):
    h = _h_of(sp_ref, a0_ref, a1_ref, inv_ref[...][:, :1])
    p_ref[...] = jnp.dot(h, wn_ref[...], preferred_element_type=jnp.float32)


def _mm_last(s_prev, agg, inv, wn_pad, interpret=False):
    return pl.pallas_call(
        _mm_last_body,
        grid=(N // BN,),
        in_specs=[_row_spec(HID), _half_spec(HID, 0), _half_spec(HID, 1),
                  _row_spec(16), _full_spec((HID, 64))],
        out_specs=_row_spec(64),
        out_shape=jax.ShapeDtypeStruct((N, 64), jnp.float32),
        interpret=interpret,
    )(s_prev, agg, agg, inv, wn_pad)


def _mm_last_s_body(sp_ref, a0_ref, a1_ref, inv_ref, ws_ref, b_ref, s_ref):
    h = _h_of(sp_ref, a0_ref, a1_ref, inv_ref[...][:, :1])
    s_ref[...] = (jnp.dot(h, ws_ref[...], preferred_element_type=jnp.float32)
                  + b_ref[...])


def _mm_last_s(s_prev, agg, inv, ws, b, interpret=False):
    return pl.pallas_call(
        _mm_last_s_body,
        grid=(N // BN,),
        in_specs=[_row_spec(HID), _half_spec(HID, 0), _half_spec(HID, 1),
                  _row_spec(16), _full_spec((HID, CLS)), _full_spec((1, CLS))],
        out_specs=_row_spec(CLS),
        out_shape=jax.ShapeDtypeStruct((N, CLS), jnp.float32),
        interpret=interpret,
    )(s_prev, agg, agg, inv, ws, b)


def _final_body(s_ref, a_ref, inv_ref, o_ref):
    a = a_ref[...]
    agg = a[:, :CLS] + a[:, 64:64 + CLS]
    o_ref[...] = s_ref[...] + agg * inv_ref[...][:, :1]


def _final(s2, agg, inv, interpret=False):
    return pl.pallas_call(
        _final_body,
        grid=(N // BN,),
        in_specs=[_row_spec(CLS), _row_spec(128), _row_spec(16)],
        out_specs=_row_spec(CLS),
        out_shape=jax.ShapeDtypeStruct((N, CLS), jnp.float32),
        interpret=interpret,
    )(s2, agg, inv)


def kernel(features, edge_index, W_self_0, W_neigh_0, b_0, W_self_1,
           W_neigh_1, b_1, W_self_2, W_neigh_2, b_2):
    e = edge_index.reshape(2, E // CHUNK, CHUNK)

    p0 = _mm_p(features, W_neigh_0)
    deg = _make_sc_deg()(e)
    agg0 = _make_sc_agg(HID)(p0, e)
    s0 = _mm_s(features, W_self_0, b_0.reshape(1, HID), HID)

    p1, inv = _mm_mid(s0, agg0, deg, W_neigh_1)
    agg1 = _make_sc_agg(HID)(p1, e)
    s1 = _mm_mid_s(s0, agg0, deg, W_self_1, b_1.reshape(1, HID))

    wn2_pad = jnp.pad(W_neigh_2, ((0, 0), (0, 64 - CLS)))
    p2 = _mm_last(s1, agg1, inv, wn2_pad)
    agg2 = _make_sc_agg(64, out_width=128)(p2, e)
    s2 = _mm_last_s(s1, agg1, inv, W_self_2, b_2.reshape(1, CLS))

    return _final(s2, agg2, inv)


# speedup vs baseline: 1.0184x; 1.0184x over previous
import functools

import jax
import jax.numpy as jnp
from jax import lax
from jax.experimental import pallas as pl
from jax.experimental.pallas import tpu as pltpu
from jax.experimental.pallas import tpu_sc as plsc

N = 10000
E = 320000
IN = 128
HID = 128
CLS = 40

NC = 2
NS = 16
NW = NC * NS
EW = E // NW
CHUNK = 80
NCHUNK = EW // CHUNK
RPT = N // NS
ZROWS = 25


def _fill(ref, rows, width, value):
    vec = jnp.full((16,), value, dtype=jnp.float32)
    per_row = width // 16

    def body(t, _):
        r = t // per_row
        col = (t % per_row) * 16
        ref[r, pl.ds(col, 16)] = vec
        return _

    lax.fori_loop(0, rows * per_row, body, None)


def _sc_mesh():
    return plsc.VectorSubcoreMesh(core_axis_name="c", subcore_axis_name="s",
                                  num_cores=NC, num_subcores=NS)


def _sc_params():
    return pltpu.CompilerParams(use_tc_tiling_on_sc=False)


def _make_sc_agg(feat_width, out_width=None, with_dep=False, interpret=False):
    scratch = [
        pltpu.VMEM((NCHUNK, CHUNK), jnp.int32),
        pltpu.VMEM((NCHUNK, CHUNK), jnp.int32),
        pltpu.VMEM((CHUNK, feat_width), jnp.float32),
        pltpu.VMEM((CHUNK, feat_width), jnp.float32),
        pltpu.VMEM((CHUNK, feat_width), jnp.float32),
        pltpu.VMEM_SHARED((N, feat_width), jnp.float32),
    ] + [pltpu.SemaphoreType.DMA] * 6
    stacked = out_width is not None
    oshape = (N, out_width) if stacked else (NC * N, feat_width)

    @functools.partial(
        pl.kernel,
        out_type=jax.ShapeDtypeStruct(oshape, jnp.float32),
        mesh=_sc_mesh(),
        scratch_types=scratch,
        compiler_params=_sc_params(),
        interpret=interpret,
    )
    def sc_agg(p_hbm, e_hbm, *refs):
        if with_dep:
            (dep_hbm, out_hbm, src_v, dst_v, rows0, rows1, rows2,
             acc, gsem0, gsem1, gsem2, ssem0, ssem1, ssem2) = refs
        else:
            (out_hbm, src_v, dst_v, rows0, rows1, rows2,
             acc, gsem0, gsem1, gsem2, ssem0, ssem1, ssem2) = refs
        c = lax.axis_index("c")
        s = lax.axis_index("s")
        w = c * NS + s
        rbase = s * RPT

        pltpu.sync_copy(e_hbm.at[0, pl.ds(w * NCHUNK, NCHUNK)], src_v)
        pltpu.sync_copy(e_hbm.at[1, pl.ds(w * NCHUNK, NCHUNK)], dst_v)

        _fill(rows0, CHUNK, feat_width, 0.0)
        for i in range(RPT // CHUNK):
            pltpu.sync_copy(rows0, acc.at[pl.ds(rbase + i * CHUNK, CHUNK)])
        rem = RPT % CHUNK
        pltpu.sync_copy(rows0.at[pl.ds(0, rem)],
                        acc.at[pl.ds(rbase + RPT - rem, rem)])
        plsc.subcore_barrier()

        bufs = (rows0, rows1, rows2)
        gsems = (gsem0, gsem1, gsem2)
        ssems = (ssem0, ssem1, ssem2)

        def gather(j, b):
            pltpu.async_copy(p_hbm.at[src_v.at[j]], bufs[b], gsems[b])

        def gwait(b):
            pltpu.make_async_copy(p_hbm.at[src_v.at[0]], bufs[b],
                                  gsems[b]).wait()

        def scat(j, b):
            pltpu.async_copy(bufs[b], acc.at[dst_v.at[j]], ssems[b], add=True)

        def swait(b):
            pltpu.make_async_copy(bufs[b], acc.at[dst_v.at[0]],
                                  ssems[b]).wait()

        def slot(j, b, wait_prev, prefetch):
            nxt = (b + 1) % 3
            if wait_prev:
                swait(nxt)
            if prefetch:
                gather(j + 1, nxt)
            gwait(b)
            scat(j, b)

        gather(0, 0)
        slot(0, 0, False, True)
        slot(1, 1, False, True)

        def body(i, _):
            slot(3 * i + 2, 2, True, True)
            slot(3 * i + 3, 0, True, True)
            slot(3 * i + 4, 1, True, True)
            return _

        lax.fori_loop(0, (NCHUNK - 5) // 3, body, None)
        slot(NCHUNK - 3, (NCHUNK - 3) % 3, True, True)
        slot(NCHUNK - 2, (NCHUNK - 2) % 3, True, True)
        slot(NCHUNK - 1, (NCHUNK - 1) % 3, True, False)
        swait((NCHUNK - 2) % 3)
        swait((NCHUNK - 1) % 3)
        plsc.subcore_barrier()

        if stacked:
            pltpu.sync_copy(
                acc.at[pl.ds(rbase, RPT)],
                out_hbm.at[pl.ds(rbase, RPT), pl.ds(c * feat_width, feat_width)])
        else:
            pltpu.sync_copy(acc.at[pl.ds(rbase, RPT)],
                            out_hbm.at[pl.ds(c * N + rbase, RPT)])

    return sc_agg


def _make_sc_deg(interpret=False):
    scratch = [
        pltpu.VMEM((NCHUNK, CHUNK), jnp.int32),
        pltpu.VMEM((CHUNK, 16), jnp.float32),
        pltpu.VMEM((ZROWS, 16), jnp.float32),
        pltpu.VMEM_SHARED((N, 16), jnp.float32),
        pltpu.SemaphoreType.DMA,
    ]

    @functools.partial(
        pl.kernel,
        out_type=jax.ShapeDtypeStruct((N, 128), jnp.float32),
        mesh=_sc_mesh(),
        scratch_types=scratch,
        compiler_params=_sc_params(),
        interpret=interpret,
    )
    def sc_deg(e_hbm, deg_hbm, dst_v, ones_v, zbuf16, dacc, sem):
        c = lax.axis_index("c")
        s = lax.axis_index("s")
        w = c * NS + s
        rbase = s * RPT

        pltpu.sync_copy(e_hbm.at[1, pl.ds(w * NCHUNK, NCHUNK)], dst_v)
        _fill(ones_v, CHUNK, 16, 1.0)
        _fill(zbuf16, ZROWS, 16, 0.0)

        def zero_body(i, _):
            pltpu.sync_copy(zbuf16, dacc.at[pl.ds(rbase + i * ZROWS, ZROWS)])
            return _

        lax.fori_loop(0, RPT // ZROWS, zero_body, None)
        plsc.subcore_barrier()

        GRP = 5

        def body(i, _):
            def fire(g, _):
                pltpu.async_copy(ones_v, dacc.at[dst_v.at[i * GRP + g]], sem,
                                 add=True)
                return _

            lax.fori_loop(0, GRP, fire, None)

            def drain(g, _):
                pltpu.make_async_copy(ones_v, dacc.at[dst_v.at[0]], sem).wait()
                return _

            lax.fori_loop(0, GRP, drain, None)
            return _

        lax.fori_loop(0, NCHUNK // GRP, body, None)
        plsc.subcore_barrier()

        pltpu.sync_copy(dacc.at[pl.ds(rbase, RPT)],
                        deg_hbm.at[pl.ds(rbase, RPT), pl.ds(c * 16, 16)])

    return sc_deg


_make_sc_agg = functools.lru_cache(maxsize=None)(_make_sc_agg)
_make_sc_deg = functools.lru_cache(maxsize=None)(_make_sc_deg)

BN = 2000


def _row_spec(width):
    return pl.BlockSpec((BN, width), lambda i: (i, 0))


def _half_spec(width, half):
    return pl.BlockSpec((BN, width), lambda i: (half * (N // BN) + i, 0))


def _full_spec(shape):
    return pl.BlockSpec(shape, lambda i: tuple(0 for _ in shape))


def _mm_p_body(x_ref, wn_ref, p_ref):
    p_ref[...] = jnp.dot(x_ref[...], wn_ref[...],
                         preferred_element_type=jnp.float32)


def _mm_p(x, wn, interpret=False):
    return pl.pallas_call(
        _mm_p_body,
        grid=(N // BN,),
        in_specs=[_row_spec(HID), _full_spec((HID, HID))],
        out_specs=_row_spec(HID),
        out_shape=jax.ShapeDtypeStruct((N, HID), jnp.float32),
        interpret=interpret,
    )(x, wn)


def _mm_s_body(x_ref, ws_ref, b_ref, s_ref):
    s_ref[...] = (jnp.dot(x_ref[...], ws_ref[...],
                          preferred_element_type=jnp.float32) + b_ref[...])


def _mm_s(x, ws, b, width, interpret=False):
    return pl.pallas_call(
        _mm_s_body,
        grid=(N // BN,),
        in_specs=[_row_spec(HID), _full_spec((HID, width)),
                  _full_spec((1, width))],
        out_specs=_row_spec(width),
        out_shape=jax.ShapeDtypeStruct((N, width), jnp.float32),
        interpret=interpret,
    )(x, ws, b)


def _h_of(sp_ref, a0_ref, a1_ref, inv):
    return jnp.maximum(sp_ref[...] + (a0_ref[...] + a1_ref[...]) * inv, 0.0)


def _inv_of(d_ref):
    deg = d_ref[...][:, :1] + d_ref[...][:, 16:17]
    return 1.0 / jnp.maximum(deg, 1.0)


def _mm_mid_body(sp_ref, a0_ref, a1_ref, d_ref, wn_ref, p_ref, inv_ref):
    inv = _inv_of(d_ref)
    h = _h_of(sp_ref, a0_ref, a1_ref, inv)
    p_ref[...] = jnp.dot(h, wn_ref[...], preferred_element_type=jnp.float32)
    inv_ref[...] = jnp.broadcast_to(inv, (BN, 16))


def _mm_mid(s_prev, agg, deg, wn, interpret=False):
    return pl.pallas_call(
        _mm_mid_body,
        grid=(N // BN,),
        in_specs=[_row_spec(HID), _half_spec(HID, 0), _half_spec(HID, 1),
                  _row_spec(128), _full_spec((HID, HID))],
        out_specs=[_row_spec(HID), _row_spec(16)],
        out_shape=[jax.ShapeDtypeStruct((N, HID), jnp.float32),
                   jax.ShapeDtypeStruct((N, 16), jnp.float32)],
        interpret=interpret,
    )(s_prev, agg, agg, deg, wn)


def _mm_mid_s_body(sp_ref, a0_ref, a1_ref, d_ref, ws_ref, b_ref, s_ref):
    h = _h_of(sp_ref, a0_ref, a1_ref, _inv_of(d_ref))
    s_ref[...] = (jnp.dot(h, ws_ref[...], preferred_element_type=jnp.float32)
                  + b_ref[...])


def _mm_mid_s(s_prev, agg, deg, ws, b, interpret=False):
    return pl.pallas_call(
        _mm_mid_s_body,
        grid=(N // BN,),
        in_specs=[_row_spec(HID), _half_spec(HID, 0), _half_spec(HID, 1),
                  _row_spec(128), _full_spec((HID, HID)),
                  _full_spec((1, HID))],
        out_specs=_row_spec(HID),
        out_shape=jax.ShapeDtypeStruct((N, HID), jnp.float32),
        interpret=interpret,
    )(s_prev, agg, agg, deg, ws, b)


def _mm_last_body(sp_ref, a0_ref, a1_ref, inv_ref, wn_ref, p_ref):
    h = _h_of(sp_ref, a0_ref, a1_ref, inv_ref[...][:, :1])
    p_ref[...] = jnp.dot(h, wn_ref[...], preferred_element_type=jnp.float32)


def _mm_last(s_prev, agg, inv, wn_pad, interpret=False):
    return pl.pallas_call(
        _mm_last_body,
        grid=(N // BN,),
        in_specs=[_row_spec(HID), _half_spec(HID, 0), _half_spec(HID, 1),
                  _row_spec(16), _full_spec((HID, 64))],
        out_specs=_row_spec(64),
        out_shape=jax.ShapeDtypeStruct((N, 64), jnp.float32),
        interpret=interpret,
    )(s_prev, agg, agg, inv, wn_pad)


def _mm_last_s_body(sp_ref, a0_ref, a1_ref, inv_ref, ws_ref, b_ref, s_ref):
    h = _h_of(sp_ref, a0_ref, a1_ref, inv_ref[...][:, :1])
    s_ref[...] = (jnp.dot(h, ws_ref[...], preferred_element_type=jnp.float32)
                  + b_ref[...])


def _mm_last_s(s_prev, agg, inv, ws, b, interpret=False):
    return pl.pallas_call(
        _mm_last_s_body,
        grid=(N // BN,),
        in_specs=[_row_spec(HID), _half_spec(HID, 0), _half_spec(HID, 1),
                  _row_spec(16), _full_spec((HID, CLS)), _full_spec((1, CLS))],
        out_specs=_row_spec(CLS),
        out_shape=jax.ShapeDtypeStruct((N, CLS), jnp.float32),
        interpret=interpret,
    )(s_prev, agg, agg, inv, ws, b)


def _final_body(s_ref, a_ref, inv_ref, o_ref):
    a = a_ref[...]
    agg = a[:, :CLS] + a[:, 64:64 + CLS]
    o_ref[...] = s_ref[...] + agg * inv_ref[...][:, :1]


def _final(s2, agg, inv, interpret=False):
    return pl.pallas_call(
        _final_body,
        grid=(N // BN,),
        in_specs=[_row_spec(CLS), _row_spec(128), _row_spec(16)],
        out_specs=_row_spec(CLS),
        out_shape=jax.ShapeDtypeStruct((N, CLS), jnp.float32),
        interpret=interpret,
    )(s2, agg, inv)


def kernel(features, edge_index, W_self_0, W_neigh_0, b_0, W_self_1,
           W_neigh_1, b_1, W_self_2, W_neigh_2, b_2):
    e = edge_index.reshape(2, E // CHUNK, CHUNK)

    p0 = _mm_p(features, W_neigh_0)
    deg = _make_sc_deg()(e)
    agg0 = _make_sc_agg(HID, with_dep=True)(p0, e, deg)
    s0 = _mm_s(features, W_self_0, b_0.reshape(1, HID), HID)

    p1, inv = _mm_mid(s0, agg0, deg, W_neigh_1)
    agg1 = _make_sc_agg(HID)(p1, e)
    s1 = _mm_mid_s(s0, agg0, deg, W_self_1, b_1.reshape(1, HID))

    wn2_pad = jnp.pad(W_neigh_2, ((0, 0), (0, 64 - CLS)))
    p2 = _mm_last(s1, agg1, inv, wn2_pad)
    agg2 = _make_sc_agg(64, out_width=128)(p2, e)
    s2 = _mm_last_s(s1, agg1, inv, W_self_2, b_2.reshape(1, CLS))

    return _final(s2, agg2, inv)
